# baseline (speedup 1.0000x reference)
import functools

import jax
import jax.numpy as jnp
from jax import lax
from jax.experimental import pallas as pl
from jax.experimental.pallas import tpu as pltpu
from jax.experimental.pallas import tpu_sc as plsc

_VOCAB = 1000000
_MAX_LEN = 200
_D = 64
_B = 4096
_L = 200

_NC = 2
_NS = 16
_NW = _NC * _NS
_N = _B * _L
_PER_W = _N // _NW
_CHUNKS = _PER_W // _L
_LANES = 16


def _sc_body(xf3, tbl, posf, out, idx_all, rows_v, pos_v, semg):
  cid = lax.axis_index("c")
  sid = lax.axis_index("s")
  wid = sid * _NC + cid
  base = wid * _PER_W

  pltpu.sync_copy(posf, pos_v)
  pltpu.sync_copy(xf3.at[wid], idx_all)

  @pl.loop(0, _CHUNKS)
  def _chunk(c):
    h1 = pltpu.async_copy(
        tbl.at[idx_all.at[c, pl.ds(0, 128)]], rows_v.at[pl.ds(0, 128)], semg)
    h2 = pltpu.async_copy(
        tbl.at[idx_all.at[c, pl.ds(128, 72)]], rows_v.at[pl.ds(128, 72)], semg)
    h1.wait()
    h2.wait()

    @pl.loop(0, _L)
    def _row(r):
      for k in range(_D // _LANES):
        v = pos_v[r, pl.ds(k * _LANES, _LANES)]
        plsc.addupdate(rows_v.at[r, pl.ds(k * _LANES, _LANES)], v)

    pltpu.sync_copy(rows_v, out.at[pl.ds(base + c * _L, _L)])


@functools.partial(jax.jit, static_argnames=())
def _run(xf3, token_table, pos_table):
  mesh = plsc.VectorSubcoreMesh(
      core_axis_name="c", subcore_axis_name="s",
      num_cores=_NC, num_subcores=_NS)
  kern = pl.kernel(
      _sc_body,
      out_type=jax.ShapeDtypeStruct((_N, _D), jnp.float32),
      mesh=mesh,
      scratch_types=[
          pltpu.VMEM((_CHUNKS, _L), jnp.int32),
          pltpu.VMEM((_L, _D), jnp.float32),
          pltpu.VMEM((_MAX_LEN, _D), jnp.float32),
          pltpu.SemaphoreType.DMA,
      ],
      compiler_params=pltpu.CompilerParams(use_tc_tiling_on_sc=False),
  )
  return kern(xf3, token_table, pos_table)


def kernel(x, token_table, pos_table):
  xf3 = x.reshape(_NW, _CHUNKS, _L)
  out = _run(xf3, token_table, pos_table[:_L])
  return out.reshape(_B, _L, _D)


# speedup vs baseline: 1.1552x; 1.1552x over previous
import functools

import jax
import jax.numpy as jnp
from jax import lax
from jax.experimental import pallas as pl
from jax.experimental.pallas import tpu as pltpu
from jax.experimental.pallas import tpu_sc as plsc

_VOCAB = 1000000
_D = 64
_B = 4096
_L = 200

_NC = 2
_NS = 16
_NW = _NC * _NS
_N = _B * _L
_PER_W = _N // _NW
_CH = 400
_NCH = _PER_W // _CH
_LANES = 16
_SPLITS = ((0, 128), (128, 128), (256, 128), (384, 16))


def _sc_body(xf3, tbl, posf, out, idx_all, rows_v, pos_v,
             gsem0, gsem1, ssem0, ssem1):
  cid = lax.axis_index("c")
  sid = lax.axis_index("s")
  wid = sid * _NC + cid
  base = wid * _PER_W

  gsems = (gsem0, gsem1)
  ssems = (ssem0, ssem1)

  pltpu.sync_copy(posf, pos_v)
  pltpu.sync_copy(xf3.at[wid], idx_all)

  def issue_gather(c, b):
    for (o, n) in _SPLITS:
      pltpu.async_copy(
          tbl.at[idx_all.at[c, pl.ds(o, n)]],
          rows_v.at[b, pl.ds(o, n)], gsems[b])

  def wait_gather(b):
    pltpu.make_async_copy(
        tbl.at[pl.ds(0, _CH)], rows_v.at[b], gsems[b]).wait()

  def issue_store(c, b):
    pltpu.async_copy(rows_v.at[b], out.at[pl.ds(base + c * _CH, _CH)],
                     ssems[b])

  def wait_store(b):
    pltpu.make_async_copy(rows_v.at[b], out.at[pl.ds(0, _CH)],
                          ssems[b]).wait()

  def add_pos(b):
    @pl.loop(0, _L, unroll=2)
    def _row(r):
      for k in range(_D // _LANES):
        v = pos_v[r, pl.ds(k * _LANES, _LANES)]
        plsc.addupdate(rows_v.at[b, r, pl.ds(k * _LANES, _LANES)], v)
        plsc.addupdate(rows_v.at[b, r + _L, pl.ds(k * _LANES, _LANES)], v)

  issue_gather(0, 0)

  @pl.loop(0, _NCH // 2)
  def _pair(c2):
    c0 = c2 * 2
    for half in range(2):
      cc = c0 + half
      b = half
      nb = 1 - half

      @pl.when(cc > 0)
      def _():
        wait_store(nb)

      @pl.when(cc + 1 < _NCH)
      def _():
        issue_gather(cc + 1, nb)

      wait_gather(b)
      add_pos(b)
      issue_store(cc, b)

  wait_store(1)


@jax.jit
def _run(xf3, token_table, pos_table):
  mesh = plsc.VectorSubcoreMesh(
      core_axis_name="c", subcore_axis_name="s",
      num_cores=_NC, num_subcores=_NS)
  kern = pl.kernel(
      _sc_body,
      out_type=jax.ShapeDtypeStruct((_N, _D), jnp.float32),
      mesh=mesh,
      scratch_types=[
          pltpu.VMEM((_NCH, _CH), jnp.int32),
          pltpu.VMEM((2, _CH, _D), jnp.float32),
          pltpu.VMEM((_L, _D), jnp.float32),
          pltpu.SemaphoreType.DMA,
          pltpu.SemaphoreType.DMA,
          pltpu.SemaphoreType.DMA,
          pltpu.SemaphoreType.DMA,
      ],
      compiler_params=pltpu.CompilerParams(use_tc_tiling_on_sc=False),
  )
  return kern(xf3, token_table, pos_table)


def kernel(x, token_table, pos_table):
  xf3 = x.reshape(_NW, _NCH, _CH)
  out = _run(xf3, token_table, pos_table[:_L])
  return out.reshape(_B, _L, _D)
